# baseline (speedup 1.0000x reference)
import functools

import jax
import jax.numpy as jnp
from jax import lax
from jax.experimental import pallas as pl
from jax.experimental.pallas import tpu as pltpu
from jax.experimental.pallas import tpu_sc as plsc

_F = 26
_V = 100000
_D = 16
_B = 16384
_NW = 32
_C = 128
_NCHUNK = _B // (_NW * _C)
_KTOT = _B // _C


def _sc_body(idx_hbm, emb_hbm, lin_hbm, par_hbm, out_hbm,
             idx_v, rows_v, lin_v, out_v, par_v, sem_e, sem_l):
    cid = lax.axis_index("c")
    sid = lax.axis_index("s")
    wid = sid * 2 + cid
    pltpu.sync_copy(par_hbm, par_v)

    def chunk(c, carry):
        k = wid * _NCHUNK + c
        pltpu.sync_copy(idx_hbm.at[k], idx_v)
        descs = []
        for j in range(_F):
            descs.append(pltpu.async_copy(
                emb_hbm.at[idx_v.at[j]], rows_v.at[pl.ds(j * _C, _C)], sem_e))
            descs.append(pltpu.async_copy(
                lin_hbm.at[idx_v.at[j]], lin_v.at[j], sem_l))
        for d in descs:
            d.wait()

        hg = par_v[pl.ds(0, 16)]
        bb = par_v[pl.ds(16, 16)]
        lanes = lax.iota(jnp.int32, 16)

        def group(j, carry2):
            qacc = jnp.zeros((16,), jnp.float32)
            sacc = [jnp.zeros((16,), jnp.float32) for _ in range(_D)]
            for f in range(_F):
                row16 = lanes + (f * _C + j * 16)
                for d in range(_D):
                    col16 = jnp.full((16,), d, jnp.int32)
                    v = plsc.load_gather(rows_v, [row16, col16])
                    sacc[d] = sacc[d] + v
                    qacc = qacc + v * v
            acc16 = -qacc
            for d in range(_D):
                acc16 = acc16 + sacc[d] * sacc[d]
            lin16 = jnp.zeros((16,), jnp.float32)
            for f in range(_F):
                lin16 = lin16 + lin_v[f, pl.ds(j * 16, 16)]
            zz = acc16 * hg + lin16 + bb
            out_v[pl.ds(j * 16, 16)] = 1.0 / (1.0 + jnp.exp(-zz))
            return carry2

        lax.fori_loop(0, _C // 16, group, 0)

        off = pl.multiple_of(k * _C, _C)
        pltpu.sync_copy(out_v, out_hbm.at[pl.ds(off, _C)])
        return carry

    lax.fori_loop(0, _NCHUNK, chunk, 0)


_sc_call = functools.partial(
    pl.kernel,
    out_type=jax.ShapeDtypeStruct((_B,), jnp.float32),
    mesh=plsc.VectorSubcoreMesh(core_axis_name="c", subcore_axis_name="s"),
    compiler_params=pltpu.CompilerParams(
        needs_layout_passes=False, use_tc_tiling_on_sc=False),
    scratch_types=[
        pltpu.VMEM((_F, _C), jnp.int32),
        pltpu.VMEM((_F * _C, _D), jnp.float32),
        pltpu.VMEM((_F, _C), jnp.float32),
        pltpu.VMEM((_C,), jnp.float32),
        pltpu.VMEM((2 * _D,), jnp.float32),
        pltpu.SemaphoreType.DMA,
        pltpu.SemaphoreType.DMA,
    ],
)(_sc_body)


@jax.jit
def kernel(x, embed_table, linear_table, bias, genotype):
    offsets = jnp.arange(_F, dtype=x.dtype) * _V
    xo = x + offsets[None, :]
    idx = xo.reshape(_KTOT, _C, _F).transpose(0, 2, 1)
    lin = linear_table.reshape(-1)
    par = jnp.concatenate([
        jnp.full((_D,), genotype[0, 0] * 0.5, jnp.float32),
        jnp.full((_D,), bias[0], jnp.float32),
    ])
    return _sc_call(idx, embed_table, lin, par)


# speedup vs baseline: 1.0261x; 1.0261x over previous
import functools

import jax
import jax.numpy as jnp
from jax import lax
from jax.experimental import pallas as pl
from jax.experimental.pallas import tpu as pltpu
from jax.experimental.pallas import tpu_sc as plsc

_F = 26
_V = 100000
_D = 16
_B = 16384
_NW = 32
_C = 128
_NCHUNK = _B // (_NW * _C)
_KTOT = _B // _C


def _sc_body(idx_hbm, emb_hbm, lin_hbm, par_hbm, out_hbm,
             idx0, idx1, rows0, rows1, lin0, lin1, out_v, par_v,
             se0, se1, sl0, sl1):
    cid = lax.axis_index("c")
    sid = lax.axis_index("s")
    wid = sid * 2 + cid
    pltpu.sync_copy(par_hbm, par_v)

    idx_b, rows_b, lin_b = (idx0, idx1), (rows0, rows1), (lin0, lin1)
    se_b, sl_b = (se0, se1), (sl0, sl1)
    descs = [None, None]

    def fire(c, b):
        k = wid * _NCHUNK + c
        pltpu.sync_copy(idx_hbm.at[k], idx_b[b])
        ds_ = []
        for j in range(_F):
            ds_.append(pltpu.async_copy(
                emb_hbm.at[idx_b[b].at[j]], rows_b[b].at[j], se_b[b]))
            ds_.append(pltpu.async_copy(
                lin_hbm.at[idx_b[b].at[j]], lin_b[b].at[j], sl_b[b]))
        descs[b] = ds_

    hg = par_v[pl.ds(0, 16)]
    bb = par_v[pl.ds(16, 16)]
    lanes = lax.iota(jnp.int32, 16)

    def compute(c, b):
        rows_v, lin_v = rows_b[b], lin_b[b]

        def group(j, carry2):
            row16 = lanes + j * 16

            def field(f, carry):
                qacc, sacc = carry
                f16 = jnp.full((16,), f, jnp.int32)
                sacc2 = []
                for d in range(_D):
                    col16 = jnp.full((16,), d, jnp.int32)
                    v = plsc.load_gather(rows_v, [f16, row16, col16])
                    sacc2.append(sacc[d] + v)
                    qacc = qacc + v * v
                return qacc, tuple(sacc2)

            zero = jnp.zeros((16,), jnp.float32)
            qacc, sacc = lax.fori_loop(
                0, _F, field, (zero, (zero,) * _D))
            acc16 = -qacc
            for d in range(_D):
                acc16 = acc16 + sacc[d] * sacc[d]
            lin16 = jnp.zeros((16,), jnp.float32)
            for f in range(_F):
                lin16 = lin16 + lin_v[f, pl.ds(j * 16, 16)]
            zz = acc16 * hg + lin16 + bb
            out_v[pl.ds(j * 16, 16)] = 1.0 / (1.0 + jnp.exp(-zz))
            return carry2

        lax.fori_loop(0, _C // 16, group, 0)
        k = wid * _NCHUNK + c
        off = pl.multiple_of(k * _C, _C)
        pltpu.sync_copy(out_v, out_hbm.at[pl.ds(off, _C)])

    fire(0, 0)
    for c in range(_NCHUNK):
        b = c & 1
        if c + 1 < _NCHUNK:
            fire(c + 1, 1 - b)
        for d in descs[b]:
            d.wait()
        compute(c, b)


_sc_call = functools.partial(
    pl.kernel,
    out_type=jax.ShapeDtypeStruct((_B,), jnp.float32),
    mesh=plsc.VectorSubcoreMesh(core_axis_name="c", subcore_axis_name="s"),
    compiler_params=pltpu.CompilerParams(
        needs_layout_passes=False, use_tc_tiling_on_sc=False),
    scratch_types=[
        pltpu.VMEM((_F, _C), jnp.int32),
        pltpu.VMEM((_F, _C), jnp.int32),
        pltpu.VMEM((_F, _C, _D), jnp.float32),
        pltpu.VMEM((_F, _C, _D), jnp.float32),
        pltpu.VMEM((_F, _C), jnp.float32),
        pltpu.VMEM((_F, _C), jnp.float32),
        pltpu.VMEM((_C,), jnp.float32),
        pltpu.VMEM((2 * _D,), jnp.float32),
        pltpu.SemaphoreType.DMA,
        pltpu.SemaphoreType.DMA,
        pltpu.SemaphoreType.DMA,
        pltpu.SemaphoreType.DMA,
    ],
)(_sc_body)


@jax.jit
def kernel(x, embed_table, linear_table, bias, genotype):
    offsets = jnp.arange(_F, dtype=x.dtype) * _V
    xo = x + offsets[None, :]
    idx = xo.reshape(_KTOT, _C, _F).transpose(0, 2, 1)
    lin = linear_table.reshape(-1)
    par = jnp.concatenate([
        jnp.full((_D,), genotype[0, 0] * 0.5, jnp.float32),
        jnp.full((_D,), bias[0], jnp.float32),
    ])
    return _sc_call(idx, embed_table, lin, par)
